# baseline (speedup 1.0000x reference)
import math

import jax
import jax.numpy as jnp
import numpy as np
from jax import lax
from jax.experimental import pallas as pl
from jax.experimental.pallas import tpu as pltpu
from jax.experimental.pallas import tpu_sc as plsc

_MAX_LEN = 100000
_TIME_DIM = 128
_BATCH = 16384

_NC = 2
_NS = 16
_NW = _NC * _NS
_L = 16

_B_TC = 4096
_B_SC = _BATCH - _B_TC
_BPW = _B_SC // _NW

_DIV_TERM = np.exp(
    np.arange(0, _TIME_DIM, 2, dtype=np.float32)
    * (-math.log(10000.0) / _TIME_DIM)
)
_DIV_FULL = np.repeat(_DIV_TERM, 2).reshape(1, _TIME_DIM)
_OFFS = np.where(
    np.arange(_TIME_DIM) % 2 == 1, np.float32(np.pi / 2), np.float32(0.0)
).astype(np.float32).reshape(1, _TIME_DIM)


def _sc_gather_body(t_hbm, pe_hbm, out_hbm, idx_v, rows_v, sem):
    wid = lax.axis_index("s") * _NC + lax.axis_index("c")
    base = wid * _BPW
    pltpu.sync_copy(t_hbm.at[pl.ds(base, _BPW)], idx_v)

    def fix(i, carry):
        s = i * _L
        v = idx_v[pl.ds(s, _L)] - 1
        idx_v[pl.ds(s, _L)] = jnp.where(v < 0, v + _MAX_LEN, v)
        return carry

    lax.fori_loop(0, _BPW // _L, fix, 0)
    pltpu.async_copy(pe_hbm.at[idx_v], rows_v, sem).wait()
    pltpu.sync_copy(rows_v, out_hbm.at[pl.ds(base, _BPW)])


def _tc_recompute_body(t_ref, df_ref, of_ref, o_ref):
    v = t_ref[...] - 1
    v = jnp.where(v < 0, v + _MAX_LEN, v)
    ang = v.astype(jnp.float32) * df_ref[...] + of_ref[...]
    o_ref[...] = jnp.sin(ang)


def kernel(t, pe):
    t32 = t.astype(jnp.int32)
    mesh = plsc.VectorSubcoreMesh(core_axis_name="c", subcore_axis_name="s")
    sc_f = pl.kernel(
        _sc_gather_body,
        mesh=mesh,
        out_type=jax.ShapeDtypeStruct((_B_SC, _TIME_DIM), jnp.float32),
        scratch_types=[
            pltpu.VMEM((_BPW,), jnp.int32),
            pltpu.VMEM((_BPW, _TIME_DIM), jnp.float32),
            pltpu.SemaphoreType.DMA,
        ],
    )
    sc_out = sc_f(t32[:_B_SC], pe)
    tc_out = pl.pallas_call(
        _tc_recompute_body,
        out_shape=jax.ShapeDtypeStruct((_B_TC, _TIME_DIM), jnp.float32),
    )(
        t32[_B_SC:].reshape(_B_TC, 1),
        jnp.asarray(_DIV_FULL),
        jnp.asarray(_OFFS),
    )
    return jnp.concatenate([sc_out, tc_out], axis=0)


# speedup vs baseline: 1.3879x; 1.3879x over previous
import jax
import jax.numpy as jnp
from jax import lax
from jax.experimental import pallas as pl
from jax.experimental.pallas import tpu as pltpu
from jax.experimental.pallas import tpu_sc as plsc

_MAX_LEN = 100000
_TIME_DIM = 128
_BATCH = 16384

_NC = 2
_NS = 16
_NW = _NC * _NS
_BPW = _BATCH // _NW
_L = 16


def _gather_body(t_hbm, pe_hbm, out_hbm, idx_v, rows_v, sem):
    wid = lax.axis_index("s") * _NC + lax.axis_index("c")
    base = wid * _BPW
    pltpu.sync_copy(t_hbm.at[pl.ds(base, _BPW)], idx_v)

    def fix(i, carry):
        s = i * _L
        v = idx_v[pl.ds(s, _L)] - 1
        idx_v[pl.ds(s, _L)] = jnp.where(v < 0, v + _MAX_LEN, v)
        return carry

    lax.fori_loop(0, _BPW // _L, fix, 0)
    pltpu.async_copy(pe_hbm.at[idx_v], rows_v, sem).wait()
    pltpu.sync_copy(rows_v, out_hbm.at[pl.ds(base, _BPW)])


def kernel(t, pe):
    t32 = t.astype(jnp.int32)
    mesh = plsc.VectorSubcoreMesh(core_axis_name="c", subcore_axis_name="s")
    f = pl.kernel(
        _gather_body,
        mesh=mesh,
        out_type=jax.ShapeDtypeStruct((_BATCH, _TIME_DIM), jnp.float32),
        scratch_types=[
            pltpu.VMEM((_BPW,), jnp.int32),
            pltpu.VMEM((_BPW, _TIME_DIM), jnp.float32),
            pltpu.SemaphoreType.DMA,
        ],
    )
    return f(t32, pe)
